# Initial kernel scaffold; baseline (speedup 1.0000x reference)
import jax
import jax.numpy as jnp
from jax.experimental import pallas as pl


def kernel(node_features, edge_index, edgetypes, W_enc, b_enc, W_gc, b_gc):
    raise NotImplementedError("write your pallas kernel here")


# speedup vs baseline: 10.6933x; 10.6933x over previous
import dataclasses
import functools

import jax
import jax.numpy as jnp
from jax import lax
from jax.experimental import pallas as pl
from jax.experimental.pallas import tpu as pltpu
from jax.experimental.pallas import tpu_sc as plsc

N = 10000
E = 320000
D = 128

NC = 2
NS = 16
NW = NC * NS

CHUNK = 128
EPT = 10240
NCHUNK = EPT // CHUNK
NPAD = 10240
RPT = NPAD // NS

BR = 1024
_HIGH = jax.lax.Precision.HIGHEST


def _sc_params():
    cp = pltpu.CompilerParams()
    if "needs_layout_passes" in pltpu.CompilerParams.__dataclass_fields__:
        cp = dataclasses.replace(cp, needs_layout_passes=False)
    return cp


def _deg_body(src_hbm, dst_hbm, hist_hbm, src_v, dst_v, hsrc_v, hdst_v):
    cid = lax.axis_index("c")
    sid = lax.axis_index("s")
    wid = cid * NS + sid

    pltpu.sync_copy(src_hbm.at[wid], src_v)
    pltpu.sync_copy(dst_hbm.at[wid], dst_v)

    zeros = jnp.zeros((16,), jnp.float32)

    @pl.loop(0, NPAD, step=16)
    def _(i):
        hsrc_v[pl.ds(i, 16)] = zeros
        hdst_v[pl.ds(i, 16)] = zeros

    ones = jnp.ones((16,), jnp.float32)

    @pl.loop(0, NCHUNK)
    def _(c):
        @pl.loop(0, CHUNK, step=16)
        def _(j):
            plsc.addupdate_scatter(hsrc_v, [src_v[c, pl.ds(j, 16)]], ones)
            plsc.addupdate_scatter(hdst_v, [dst_v[c, pl.ds(j, 16)]], ones)

    pltpu.sync_copy(hsrc_v, hist_hbm.at[wid, 0])
    pltpu.sync_copy(hdst_v, hist_hbm.at[wid, 1])


def _degrees(src_p, dst_p):
    mesh = plsc.VectorSubcoreMesh(core_axis_name="c", subcore_axis_name="s")
    return pl.kernel(
        _deg_body,
        out_type=jax.ShapeDtypeStruct((NW, 2, NPAD), jnp.float32),
        mesh=mesh,
        scratch_types=[
            pltpu.VMEM((NCHUNK, CHUNK), jnp.int32),
            pltpu.VMEM((NCHUNK, CHUNK), jnp.int32),
            pltpu.VMEM((NPAD,), jnp.float32),
            pltpu.VMEM((NPAD,), jnp.float32),
        ],
        compiler_params=_sc_params(),
    )(src_p, dst_p)


def _enc_body(x_ref, w_ref, b_ref, hist_ref, h0_ref, hs_ref, nd_ref):
    h0 = jnp.dot(x_ref[...], w_ref[...], preferred_element_type=jnp.float32,
                 precision=_HIGH)
    h0 = jnp.maximum(h0 + b_ref[...][None, :], 0.0)
    degs = jnp.sum(hist_ref[...], axis=0) + 1.0
    ns = jax.lax.rsqrt(degs[0])
    nd = jax.lax.rsqrt(degs[1])
    h0_ref[...] = h0
    hs_ref[...] = h0 * ns[:, None]
    nd_ref[...] = nd


def _encode(x_pad, w_enc, b_enc, hist):
    grid = (NPAD // BR,)
    return pl.pallas_call(
        _enc_body,
        grid=grid,
        in_specs=[
            pl.BlockSpec((BR, D), lambda i: (i, 0)),
            pl.BlockSpec((D, D), lambda i: (0, 0)),
            pl.BlockSpec((D,), lambda i: (0,)),
            pl.BlockSpec((NW, 2, BR), lambda i: (0, 0, i)),
        ],
        out_specs=[
            pl.BlockSpec((BR, D), lambda i: (i, 0)),
            pl.BlockSpec((BR, D), lambda i: (i, 0)),
            pl.BlockSpec((BR,), lambda i: (i,)),
        ],
        out_shape=[
            jax.ShapeDtypeStruct((NPAD, D), jnp.float32),
            jax.ShapeDtypeStruct((NPAD, D), jnp.float32),
            jax.ShapeDtypeStruct((NPAD,), jnp.float32),
        ],
    )(x_pad, w_enc, b_enc, hist)


def _msg_body(hs_hbm, src_hbm, dst_hbm, part_hbm, src_v, dst_v, buf_v, agg_sh):
    cid = lax.axis_index("c")
    sid = lax.axis_index("s")
    wid = cid * NS + sid

    pltpu.sync_copy(src_hbm.at[wid], src_v)
    pltpu.sync_copy(dst_hbm.at[wid], dst_v)

    zeros = jnp.zeros((16,), jnp.float32)

    @pl.loop(0, CHUNK)
    def _(r):
        @pl.loop(0, D, step=16)
        def _(j):
            buf_v[r, pl.ds(j, 16)] = zeros

    r0 = sid * RPT

    @pl.loop(0, RPT, step=CHUNK)
    def _(k):
        pltpu.sync_copy(buf_v, agg_sh.at[pl.ds(r0 + k, CHUNK)])

    plsc.subcore_barrier()

    @pl.loop(0, NCHUNK)
    def _(c):
        pltpu.sync_copy(hs_hbm.at[src_v.at[c]], buf_v)
        pltpu.sync_copy(buf_v, agg_sh.at[dst_v.at[c]], add=True)

    plsc.subcore_barrier()

    pltpu.sync_copy(agg_sh.at[pl.ds(r0, RPT)], part_hbm.at[cid, pl.ds(r0, RPT)])


def _messages(hs_pad, src_p, dst_p):
    mesh = plsc.VectorSubcoreMesh(core_axis_name="c", subcore_axis_name="s")
    return pl.kernel(
        _msg_body,
        out_type=jax.ShapeDtypeStruct((NC, NPAD, D), jnp.float32),
        mesh=mesh,
        scratch_types=[
            pltpu.VMEM((NCHUNK, CHUNK), jnp.int32),
            pltpu.VMEM((NCHUNK, CHUNK), jnp.int32),
            pltpu.VMEM((CHUNK, D), jnp.float32),
            pltpu.VMEM_SHARED((NPAD, D), jnp.float32),
        ],
        compiler_params=_sc_params(),
    )(hs_pad, src_p, dst_p)


def _fin_body(part_ref, hs_ref, h0_ref, nd_ref, w_ref, b_ref, out_ref):
    p = part_ref[...]
    agg = p[0] + p[1] + hs_ref[...]
    a1 = agg * nd_ref[...][:, None]
    h1 = jnp.dot(a1, w_ref[...], preferred_element_type=jnp.float32,
                 precision=_HIGH)
    out_ref[...] = h0_ref[...] + h1 + b_ref[...][None, :]


def _finalize(part, hs_pad, h0_pad, nd, w_gc, b_gc):
    grid = (NPAD // BR,)
    return pl.pallas_call(
        _fin_body,
        grid=grid,
        in_specs=[
            pl.BlockSpec((NC, BR, D), lambda i: (0, i, 0)),
            pl.BlockSpec((BR, D), lambda i: (i, 0)),
            pl.BlockSpec((BR, D), lambda i: (i, 0)),
            pl.BlockSpec((BR,), lambda i: (i,)),
            pl.BlockSpec((D, D), lambda i: (0, 0)),
            pl.BlockSpec((D,), lambda i: (0,)),
        ],
        out_specs=pl.BlockSpec((BR, D), lambda i: (i, 0)),
        out_shape=jax.ShapeDtypeStruct((NPAD, D), jnp.float32),
    )(part, hs_pad, h0_pad, nd, w_gc, b_gc)


def kernel(node_features, edge_index, edgetypes, W_enc, b_enc, W_gc, b_gc):
    del edgetypes
    dummy = (N + jnp.arange(EPT - E // NW, dtype=jnp.int32))[None, :]
    dummy = jnp.broadcast_to(dummy, (NW, EPT - E // NW))
    src_p = jnp.concatenate(
        [edge_index[0].reshape(NW, E // NW), dummy], axis=1
    ).reshape(NW, NCHUNK, CHUNK)
    dst_p = jnp.concatenate(
        [edge_index[1].reshape(NW, E // NW), dummy], axis=1
    ).reshape(NW, NCHUNK, CHUNK)
    x_pad = jnp.pad(node_features, ((0, NPAD - N), (0, 0)))

    hist = _degrees(src_p, dst_p)
    h0_pad, hs_pad, nd = _encode(x_pad, W_enc, b_enc, hist)
    part = _messages(hs_pad, src_p, dst_p)
    out_pad = _finalize(part, hs_pad, h0_pad, nd, W_gc, b_gc)
    return out_pad[:N]


# speedup vs baseline: 14.4313x; 1.3496x over previous
import dataclasses
import functools

import jax
import jax.numpy as jnp
from jax import lax
from jax.experimental import pallas as pl
from jax.experimental.pallas import tpu as pltpu
from jax.experimental.pallas import tpu_sc as plsc

N = 10000
E = 320000
D = 128

NC = 2
NS = 16
NW = NC * NS

CHUNK = 128
EPT = 10240
NCHUNK = EPT // CHUNK
NHALF = 2
HCH = NCHUNK // NHALF
NPAD = 10240
RPT = NPAD // NS

BR = 1024
_HIGH = jax.lax.Precision.HIGHEST


def _sc_params():
    cp = pltpu.CompilerParams()
    if "needs_layout_passes" in pltpu.CompilerParams.__dataclass_fields__:
        cp = dataclasses.replace(cp, needs_layout_passes=False)
    return cp


def _deg_body(src_hbm, dst_hbm, hist_hbm, src_v, dst_v, hsrc_v, hdst_v):
    cid = lax.axis_index("c")
    sid = lax.axis_index("s")
    wid = cid * NS + sid

    pltpu.sync_copy(src_hbm.at[wid], src_v)
    pltpu.sync_copy(dst_hbm.at[wid], dst_v)

    zeros = jnp.zeros((16,), jnp.float32)

    @pl.loop(0, NPAD, step=16)
    def _(i):
        hsrc_v[pl.ds(i, 16)] = zeros
        hdst_v[pl.ds(i, 16)] = zeros

    ones = jnp.ones((16,), jnp.float32)

    @pl.loop(0, NHALF)
    def _(h):
        @pl.loop(0, HCH)
        def _(c):
            @pl.loop(0, CHUNK, step=16)
            def _(j):
                plsc.addupdate_scatter(hsrc_v, [src_v[h, c, pl.ds(j, 16)]],
                                       ones)
                plsc.addupdate_scatter(hdst_v, [dst_v[h, c, pl.ds(j, 16)]],
                                       ones)

    pltpu.sync_copy(hsrc_v, hist_hbm.at[wid, 0])
    pltpu.sync_copy(hdst_v, hist_hbm.at[wid, 1])


def _degrees(src_p, dst_p):
    mesh = plsc.VectorSubcoreMesh(core_axis_name="c", subcore_axis_name="s")
    return pl.kernel(
        _deg_body,
        out_type=jax.ShapeDtypeStruct((NW, 2, NPAD), jnp.float32),
        mesh=mesh,
        scratch_types=[
            pltpu.VMEM((NHALF, HCH, CHUNK), jnp.int32),
            pltpu.VMEM((NHALF, HCH, CHUNK), jnp.int32),
            pltpu.VMEM((NPAD,), jnp.float32),
            pltpu.VMEM((NPAD,), jnp.float32),
        ],
        compiler_params=_sc_params(),
    )(src_p, dst_p)


def _enc_body(x_ref, w_ref, b_ref, hist_ref, h0_ref, hs_ref, nd_ref):
    h0 = jnp.dot(x_ref[...], w_ref[...], preferred_element_type=jnp.float32,
                 precision=_HIGH)
    h0 = jnp.maximum(h0 + b_ref[...][None, :], 0.0)
    degs = jnp.sum(hist_ref[...], axis=0) + 1.0
    ns = jax.lax.rsqrt(degs[0])
    nd = jax.lax.rsqrt(degs[1])
    h0_ref[...] = h0
    hs_ref[...] = h0 * ns[:, None]
    nd_ref[...] = nd


def _encode(x_pad, w_enc, b_enc, hist):
    grid = (NPAD // BR,)
    return pl.pallas_call(
        _enc_body,
        grid=grid,
        in_specs=[
            pl.BlockSpec((BR, D), lambda i: (i, 0)),
            pl.BlockSpec((D, D), lambda i: (0, 0)),
            pl.BlockSpec((D,), lambda i: (0,)),
            pl.BlockSpec((NW, 2, BR), lambda i: (0, 0, i)),
        ],
        out_specs=[
            pl.BlockSpec((BR, D), lambda i: (i, 0)),
            pl.BlockSpec((BR, D), lambda i: (i, 0)),
            pl.BlockSpec((BR,), lambda i: (i,)),
        ],
        out_shape=[
            jax.ShapeDtypeStruct((NPAD, D), jnp.float32),
            jax.ShapeDtypeStruct((NPAD, D), jnp.float32),
            jax.ShapeDtypeStruct((NPAD,), jnp.float32),
        ],
    )(x_pad, w_enc, b_enc, hist)


NBUF = 2


def _msg_body(hs_hbm, src_hbm, dst_hbm, part_hbm, src_v, dst_v, buf_v, agg_sh,
              gs0, gs1, ss0, ss1):
    gsems = (gs0, gs1)
    ssems = (ss0, ss1)
    cid = lax.axis_index("c")
    sid = lax.axis_index("s")
    wid = cid * NS + sid

    zeros = jnp.zeros((16,), jnp.float32)

    @pl.loop(0, CHUNK)
    def _(r):
        @pl.loop(0, D, step=16)
        def _(j):
            buf_v[0, r, pl.ds(j, 16)] = zeros

    r0 = sid * RPT

    @pl.loop(0, RPT, step=CHUNK)
    def _(k):
        pltpu.sync_copy(buf_v.at[0], agg_sh.at[pl.ds(r0 + k, CHUNK)])

    plsc.subcore_barrier()

    def _gather(cc, b):
        return pltpu.make_async_copy(hs_hbm.at[src_v.at[cc]], buf_v.at[b],
                                     gsems[b])

    def _scatter(cc, b):
        return pltpu.make_async_copy(buf_v.at[b], agg_sh.at[dst_v.at[cc]],
                                     ssems[b])

    for half in range(NHALF):
        pltpu.sync_copy(src_hbm.at[wid, half], src_v)
        pltpu.sync_copy(dst_hbm.at[wid, half], dst_v)

        for b in range(NBUF):
            pltpu.async_copy(hs_hbm.at[src_v.at[b]], buf_v.at[b], gsems[b])

        @pl.loop(0, HCH - NBUF, step=NBUF)
        def _(c):
            for b in range(NBUF):
                cc = c + b
                _gather(cc, b).wait()
                pltpu.async_copy(buf_v.at[b], agg_sh.at[dst_v.at[cc]],
                                 ssems[b], add=True)
                _scatter(cc, b).wait()
                pltpu.async_copy(hs_hbm.at[src_v.at[cc + NBUF]], buf_v.at[b],
                                 gsems[b])

        for b in range(NBUF):
            cc = HCH - NBUF + b
            _gather(cc, b).wait()
            pltpu.async_copy(buf_v.at[b], agg_sh.at[dst_v.at[cc]], ssems[b],
                             add=True)
            _scatter(cc, b).wait()

    plsc.subcore_barrier()

    pltpu.sync_copy(agg_sh.at[pl.ds(r0, RPT)], part_hbm.at[cid, pl.ds(r0, RPT)])


def _messages(hs_pad, src_p, dst_p):
    mesh = plsc.VectorSubcoreMesh(core_axis_name="c", subcore_axis_name="s")
    return pl.kernel(
        _msg_body,
        out_type=jax.ShapeDtypeStruct((NC, NPAD, D), jnp.float32),
        mesh=mesh,
        scratch_types=[
            pltpu.VMEM((HCH, CHUNK), jnp.int32),
            pltpu.VMEM((HCH, CHUNK), jnp.int32),
            pltpu.VMEM((NBUF, CHUNK, D), jnp.float32),
            pltpu.VMEM_SHARED((NPAD, D), jnp.float32),
            pltpu.SemaphoreType.DMA,
            pltpu.SemaphoreType.DMA,
            pltpu.SemaphoreType.DMA,
            pltpu.SemaphoreType.DMA,
        ],
        compiler_params=_sc_params(),
    )(hs_pad, src_p, dst_p)


def _fin_body(part_ref, hs_ref, h0_ref, nd_ref, w_ref, b_ref, out_ref):
    p = part_ref[...]
    agg = p[0] + p[1] + hs_ref[...]
    a1 = agg * nd_ref[...][:, None]
    h1 = jnp.dot(a1, w_ref[...], preferred_element_type=jnp.float32,
                 precision=_HIGH)
    out_ref[...] = h0_ref[...] + h1 + b_ref[...][None, :]


def _finalize(part, hs_pad, h0_pad, nd, w_gc, b_gc):
    grid = (NPAD // BR,)
    return pl.pallas_call(
        _fin_body,
        grid=grid,
        in_specs=[
            pl.BlockSpec((NC, BR, D), lambda i: (0, i, 0)),
            pl.BlockSpec((BR, D), lambda i: (i, 0)),
            pl.BlockSpec((BR, D), lambda i: (i, 0)),
            pl.BlockSpec((BR,), lambda i: (i,)),
            pl.BlockSpec((D, D), lambda i: (0, 0)),
            pl.BlockSpec((D,), lambda i: (0,)),
        ],
        out_specs=pl.BlockSpec((BR, D), lambda i: (i, 0)),
        out_shape=jax.ShapeDtypeStruct((NPAD, D), jnp.float32),
    )(part, hs_pad, h0_pad, nd, w_gc, b_gc)


def kernel(node_features, edge_index, edgetypes, W_enc, b_enc, W_gc, b_gc):
    del edgetypes
    dummy = (N + jnp.arange(EPT - E // NW, dtype=jnp.int32))[None, :]
    dummy = jnp.broadcast_to(dummy, (NW, EPT - E // NW))
    src_p = jnp.concatenate(
        [edge_index[0].reshape(NW, E // NW), dummy], axis=1
    ).reshape(NW, NHALF, HCH, CHUNK)
    dst_p = jnp.concatenate(
        [edge_index[1].reshape(NW, E // NW), dummy], axis=1
    ).reshape(NW, NHALF, HCH, CHUNK)
    x_pad = jnp.pad(node_features, ((0, NPAD - N), (0, 0)))

    hist = _degrees(src_p, dst_p)
    h0_pad, hs_pad, nd = _encode(x_pad, W_enc, b_enc, hist)
    part = _messages(hs_pad, src_p, dst_p)
    out_pad = _finalize(part, hs_pad, h0_pad, nd, W_gc, b_gc)
    return out_pad[:N]


# speedup vs baseline: 14.9979x; 1.0393x over previous
import dataclasses
import functools

import jax
import jax.numpy as jnp
from jax import lax
from jax.experimental import pallas as pl
from jax.experimental.pallas import tpu as pltpu
from jax.experimental.pallas import tpu_sc as plsc

N = 10000
E = 320000
D = 128

NC = 2
NS = 16
NW = NC * NS

CHUNK = 128
EPT = 10240
NCHUNK = EPT // CHUNK
NHALF = 2
HCH = NCHUNK // NHALF
NPAD = 10240
RPT = NPAD // NS

BR = 1024
_HIGH = jax.lax.Precision.DEFAULT


def _sc_params():
    cp = pltpu.CompilerParams()
    if "needs_layout_passes" in pltpu.CompilerParams.__dataclass_fields__:
        cp = dataclasses.replace(cp, needs_layout_passes=False)
    return cp


def _deg_body(src_hbm, dst_hbm, hist_hbm, src_v, dst_v, hsrc_v, hdst_v):
    cid = lax.axis_index("c")
    sid = lax.axis_index("s")
    wid = cid * NS + sid

    pltpu.sync_copy(src_hbm.at[wid], src_v)
    pltpu.sync_copy(dst_hbm.at[wid], dst_v)

    zeros = jnp.zeros((16,), jnp.float32)

    @pl.loop(0, NPAD, step=16)
    def _(i):
        hsrc_v[pl.ds(i, 16)] = zeros
        hdst_v[pl.ds(i, 16)] = zeros

    ones = jnp.ones((16,), jnp.float32)

    @pl.loop(0, NHALF)
    def _(h):
        @pl.loop(0, HCH)
        def _(c):
            @pl.loop(0, CHUNK, step=16)
            def _(j):
                plsc.addupdate_scatter(hsrc_v, [src_v[h, c, pl.ds(j, 16)]],
                                       ones)
                plsc.addupdate_scatter(hdst_v, [dst_v[h, c, pl.ds(j, 16)]],
                                       ones)

    pltpu.sync_copy(hsrc_v, hist_hbm.at[wid, 0])
    pltpu.sync_copy(hdst_v, hist_hbm.at[wid, 1])


def _degrees(src_p, dst_p):
    mesh = plsc.VectorSubcoreMesh(core_axis_name="c", subcore_axis_name="s")
    return pl.kernel(
        _deg_body,
        out_type=jax.ShapeDtypeStruct((NW, 2, NPAD), jnp.float32),
        mesh=mesh,
        scratch_types=[
            pltpu.VMEM((NHALF, HCH, CHUNK), jnp.int32),
            pltpu.VMEM((NHALF, HCH, CHUNK), jnp.int32),
            pltpu.VMEM((NPAD,), jnp.float32),
            pltpu.VMEM((NPAD,), jnp.float32),
        ],
        compiler_params=_sc_params(),
    )(src_p, dst_p)


def _enc_body(x_ref, w_ref, b_ref, hist_ref, h0_ref, hs_ref, nd_ref):
    h0 = jnp.dot(x_ref[...], w_ref[...], preferred_element_type=jnp.float32,
                 precision=_HIGH)
    h0 = jnp.maximum(h0 + b_ref[...][None, :], 0.0)
    degs = jnp.sum(hist_ref[...], axis=0) + 1.0
    ns = jax.lax.rsqrt(degs[0])
    nd = jax.lax.rsqrt(degs[1])
    h0_ref[...] = h0
    hs_ref[...] = h0 * ns[:, None]
    nd_ref[...] = nd


def _encode(x_pad, w_enc, b_enc, hist):
    grid = (NPAD // BR,)
    return pl.pallas_call(
        _enc_body,
        grid=grid,
        in_specs=[
            pl.BlockSpec((BR, D), lambda i: (i, 0)),
            pl.BlockSpec((D, D), lambda i: (0, 0)),
            pl.BlockSpec((D,), lambda i: (0,)),
            pl.BlockSpec((NW, 2, BR), lambda i: (0, 0, i)),
        ],
        out_specs=[
            pl.BlockSpec((BR, D), lambda i: (i, 0)),
            pl.BlockSpec((BR, D), lambda i: (i, 0)),
            pl.BlockSpec((BR,), lambda i: (i,)),
        ],
        out_shape=[
            jax.ShapeDtypeStruct((NPAD, D), jnp.float32),
            jax.ShapeDtypeStruct((NPAD, D), jnp.float32),
            jax.ShapeDtypeStruct((NPAD,), jnp.float32),
        ],
    )(x_pad, w_enc, b_enc, hist)


NBUF = 2


def _msg_body(hs_hbm, src_hbm, dst_hbm, part_hbm, src_v, dst_v, buf_v, agg_sh,
              gs0, gs1, ss0, ss1):
    gsems = (gs0, gs1)
    ssems = (ss0, ss1)
    cid = lax.axis_index("c")
    sid = lax.axis_index("s")
    wid = cid * NS + sid

    zeros = jnp.zeros((16,), jnp.float32)

    @pl.loop(0, CHUNK)
    def _(r):
        @pl.loop(0, D, step=16)
        def _(j):
            buf_v[0, r, pl.ds(j, 16)] = zeros

    r0 = sid * RPT

    @pl.loop(0, RPT, step=CHUNK)
    def _(k):
        pltpu.sync_copy(buf_v.at[0], agg_sh.at[pl.ds(r0 + k, CHUNK)])

    plsc.subcore_barrier()

    def _gather(cc, b):
        return pltpu.make_async_copy(hs_hbm.at[src_v.at[cc]], buf_v.at[b],
                                     gsems[b])

    def _scatter(cc, b):
        return pltpu.make_async_copy(buf_v.at[b], agg_sh.at[dst_v.at[cc]],
                                     ssems[b])

    for half in range(NHALF):
        pltpu.sync_copy(src_hbm.at[wid, half], src_v)
        pltpu.sync_copy(dst_hbm.at[wid, half], dst_v)

        for b in range(NBUF):
            pltpu.async_copy(hs_hbm.at[src_v.at[b]], buf_v.at[b], gsems[b])

        @pl.loop(0, HCH - NBUF, step=NBUF)
        def _(c):
            for b in range(NBUF):
                cc = c + b
                _gather(cc, b).wait()
                pltpu.async_copy(buf_v.at[b], agg_sh.at[dst_v.at[cc]],
                                 ssems[b], add=True)
                _scatter(cc, b).wait()
                pltpu.async_copy(hs_hbm.at[src_v.at[cc + NBUF]], buf_v.at[b],
                                 gsems[b])

        for b in range(NBUF):
            cc = HCH - NBUF + b
            _gather(cc, b).wait()
            pltpu.async_copy(buf_v.at[b], agg_sh.at[dst_v.at[cc]], ssems[b],
                             add=True)
            _scatter(cc, b).wait()

    plsc.subcore_barrier()

    pltpu.sync_copy(agg_sh.at[pl.ds(r0, RPT)], part_hbm.at[cid, pl.ds(r0, RPT)])


def _messages(hs_pad, src_p, dst_p):
    mesh = plsc.VectorSubcoreMesh(core_axis_name="c", subcore_axis_name="s")
    return pl.kernel(
        _msg_body,
        out_type=jax.ShapeDtypeStruct((NC, NPAD, D), jnp.float32),
        mesh=mesh,
        scratch_types=[
            pltpu.VMEM((HCH, CHUNK), jnp.int32),
            pltpu.VMEM((HCH, CHUNK), jnp.int32),
            pltpu.VMEM((NBUF, CHUNK, D), jnp.float32),
            pltpu.VMEM_SHARED((NPAD, D), jnp.float32),
            pltpu.SemaphoreType.DMA,
            pltpu.SemaphoreType.DMA,
            pltpu.SemaphoreType.DMA,
            pltpu.SemaphoreType.DMA,
        ],
        compiler_params=_sc_params(),
    )(hs_pad, src_p, dst_p)


def _fin_body(part_ref, hs_ref, h0_ref, nd_ref, w_ref, b_ref, out_ref):
    p = part_ref[...]
    agg = p[0] + p[1] + hs_ref[...]
    a1 = agg * nd_ref[...][:, None]
    h1 = jnp.dot(a1, w_ref[...], preferred_element_type=jnp.float32,
                 precision=_HIGH)
    out_ref[...] = h0_ref[...] + h1 + b_ref[...][None, :]


def _finalize(part, hs_pad, h0_pad, nd, w_gc, b_gc):
    grid = (NPAD // BR,)
    return pl.pallas_call(
        _fin_body,
        grid=grid,
        in_specs=[
            pl.BlockSpec((NC, BR, D), lambda i: (0, i, 0)),
            pl.BlockSpec((BR, D), lambda i: (i, 0)),
            pl.BlockSpec((BR, D), lambda i: (i, 0)),
            pl.BlockSpec((BR,), lambda i: (i,)),
            pl.BlockSpec((D, D), lambda i: (0, 0)),
            pl.BlockSpec((D,), lambda i: (0,)),
        ],
        out_specs=pl.BlockSpec((BR, D), lambda i: (i, 0)),
        out_shape=jax.ShapeDtypeStruct((N, D), jnp.float32),
    )(part, hs_pad, h0_pad, nd, w_gc, b_gc)


def kernel(node_features, edge_index, edgetypes, W_enc, b_enc, W_gc, b_gc):
    del edgetypes
    dummy = (N + jnp.arange(EPT - E // NW, dtype=jnp.int32))[None, :]
    dummy = jnp.broadcast_to(dummy, (NW, EPT - E // NW))
    src_p = jnp.concatenate(
        [edge_index[0].reshape(NW, E // NW), dummy], axis=1
    ).reshape(NW, NHALF, HCH, CHUNK)
    dst_p = jnp.concatenate(
        [edge_index[1].reshape(NW, E // NW), dummy], axis=1
    ).reshape(NW, NHALF, HCH, CHUNK)
    x_pad = jnp.pad(node_features, ((0, NPAD - N), (0, 0)))

    hist = _degrees(src_p, dst_p)
    h0_pad, hs_pad, nd = _encode(x_pad, W_enc, b_enc, hist)
    part = _messages(hs_pad, src_p, dst_p)
    return _finalize(part, hs_pad, h0_pad, nd, W_gc, b_gc)


# speedup vs baseline: 15.1750x; 1.0118x over previous
import dataclasses
import functools

import jax
import jax.numpy as jnp
from jax import lax
from jax.experimental import pallas as pl
from jax.experimental.pallas import tpu as pltpu
from jax.experimental.pallas import tpu_sc as plsc

N = 10000
E = 320000
D = 128

NC = 2
NS = 16
NW = NC * NS

CHUNK = 128
EPT = 10240
NCHUNK = EPT // CHUNK
NHALF = 2
HCH = NCHUNK // NHALF
NPAD = 10240
RPT = NPAD // NS

BR = 1024
_HIGH = jax.lax.Precision.DEFAULT


def _sc_params():
    cp = pltpu.CompilerParams()
    if "needs_layout_passes" in pltpu.CompilerParams.__dataclass_fields__:
        cp = dataclasses.replace(cp, needs_layout_passes=False)
    return cp


def _deg_body(src_hbm, dst_hbm, hist_hbm, src_v, dst_v, hsrc_v, hdst_v):
    cid = lax.axis_index("c")
    sid = lax.axis_index("s")
    wid = cid * NS + sid

    pltpu.sync_copy(src_hbm.at[wid], src_v)
    pltpu.sync_copy(dst_hbm.at[wid], dst_v)

    zeros = jnp.zeros((16,), jnp.float32)

    @pl.loop(0, NPAD, step=16)
    def _(i):
        hsrc_v[pl.ds(i, 16)] = zeros
        hdst_v[pl.ds(i, 16)] = zeros

    ones = jnp.ones((16,), jnp.float32)

    @pl.loop(0, NHALF)
    def _(h):
        @pl.loop(0, HCH)
        def _(c):
            @pl.loop(0, CHUNK, step=16)
            def _(j):
                plsc.addupdate_scatter(hsrc_v, [src_v[h, c, pl.ds(j, 16)]],
                                       ones)
                plsc.addupdate_scatter(hdst_v, [dst_v[h, c, pl.ds(j, 16)]],
                                       ones)

    pltpu.sync_copy(hsrc_v, hist_hbm.at[wid, 0])
    pltpu.sync_copy(hdst_v, hist_hbm.at[wid, 1])


def _degrees(src_p, dst_p):
    mesh = plsc.VectorSubcoreMesh(core_axis_name="c", subcore_axis_name="s")
    return pl.kernel(
        _deg_body,
        out_type=jax.ShapeDtypeStruct((NW, 2, NPAD), jnp.float32),
        mesh=mesh,
        scratch_types=[
            pltpu.VMEM((NHALF, HCH, CHUNK), jnp.int32),
            pltpu.VMEM((NHALF, HCH, CHUNK), jnp.int32),
            pltpu.VMEM((NPAD,), jnp.float32),
            pltpu.VMEM((NPAD,), jnp.float32),
        ],
        compiler_params=_sc_params(),
    )(src_p, dst_p)


def _enc_body(x_ref, w_ref, b_ref, h0_ref):
    h0 = jnp.dot(x_ref[...], w_ref[...], preferred_element_type=jnp.float32,
                 precision=_HIGH)
    h0_ref[...] = jnp.maximum(h0 + b_ref[...][None, :], 0.0)


def _ench0(x, w_enc, b_enc):
    grid = (NPAD // BR,)
    return pl.pallas_call(
        _enc_body,
        grid=grid,
        in_specs=[
            pl.BlockSpec((BR, D), lambda i: (i, 0)),
            pl.BlockSpec((D, D), lambda i: (0, 0)),
            pl.BlockSpec((D,), lambda i: (0,)),
        ],
        out_specs=pl.BlockSpec((BR, D), lambda i: (i, 0)),
        out_shape=jax.ShapeDtypeStruct((N, D), jnp.float32),
    )(x, w_enc, b_enc)


def _scale_body(h0_ref, hist_ref, hs_ref, nd_ref, nsnd_ref):
    i = pl.program_id(0)
    degs = jnp.sum(hist_ref[...], axis=0) + 1.0
    ns = jax.lax.rsqrt(degs[0])
    nd = jax.lax.rsqrt(degs[1])
    rows = jax.lax.broadcasted_iota(jnp.int32, (BR, D), 0) + i * BR
    hs_ref[...] = jnp.where(rows < N, h0_ref[...] * ns[:, None], 0.0)
    nd_ref[...] = nd
    nsnd_ref[...] = ns * nd


def _scale(h0, hist):
    grid = (NPAD // BR,)
    return pl.pallas_call(
        _scale_body,
        grid=grid,
        in_specs=[
            pl.BlockSpec((BR, D), lambda i: (i, 0)),
            pl.BlockSpec((NW, 2, BR), lambda i: (0, 0, i)),
        ],
        out_specs=[
            pl.BlockSpec((BR, D), lambda i: (i, 0)),
            pl.BlockSpec((BR,), lambda i: (i,)),
            pl.BlockSpec((BR,), lambda i: (i,)),
        ],
        out_shape=[
            jax.ShapeDtypeStruct((NPAD, D), jnp.float32),
            jax.ShapeDtypeStruct((NPAD,), jnp.float32),
            jax.ShapeDtypeStruct((NPAD,), jnp.float32),
        ],
    )(h0, hist)


NBUF = 2


def _msg_body(hs_hbm, src_hbm, dst_hbm, part_hbm, src_v, dst_v, buf_v, agg_sh,
              gs0, gs1, ss0, ss1):
    gsems = (gs0, gs1)
    ssems = (ss0, ss1)
    cid = lax.axis_index("c")
    sid = lax.axis_index("s")
    wid = cid * NS + sid

    zeros = jnp.zeros((16,), jnp.float32)

    @pl.loop(0, CHUNK)
    def _(r):
        @pl.loop(0, D, step=16)
        def _(j):
            buf_v[0, r, pl.ds(j, 16)] = zeros

    r0 = sid * RPT

    @pl.loop(0, RPT, step=CHUNK)
    def _(k):
        pltpu.sync_copy(buf_v.at[0], agg_sh.at[pl.ds(r0 + k, CHUNK)])

    plsc.subcore_barrier()

    def _gather(cc, b):
        return pltpu.make_async_copy(hs_hbm.at[src_v.at[cc]], buf_v.at[b],
                                     gsems[b])

    def _scatter(cc, b):
        return pltpu.make_async_copy(buf_v.at[b], agg_sh.at[dst_v.at[cc]],
                                     ssems[b])

    for half in range(NHALF):
        pltpu.sync_copy(src_hbm.at[wid, half], src_v)
        pltpu.sync_copy(dst_hbm.at[wid, half], dst_v)

        for b in range(NBUF):
            pltpu.async_copy(hs_hbm.at[src_v.at[b]], buf_v.at[b], gsems[b])

        @pl.loop(0, HCH - NBUF, step=NBUF)
        def _(c):
            for b in range(NBUF):
                cc = c + b
                _gather(cc, b).wait()
                pltpu.async_copy(buf_v.at[b], agg_sh.at[dst_v.at[cc]],
                                 ssems[b], add=True)
                _scatter(cc, b).wait()
                pltpu.async_copy(hs_hbm.at[src_v.at[cc + NBUF]], buf_v.at[b],
                                 gsems[b])

        for b in range(NBUF):
            cc = HCH - NBUF + b
            _gather(cc, b).wait()
            pltpu.async_copy(buf_v.at[b], agg_sh.at[dst_v.at[cc]], ssems[b],
                             add=True)
            _scatter(cc, b).wait()

    plsc.subcore_barrier()

    pltpu.sync_copy(agg_sh.at[pl.ds(r0, RPT)], part_hbm.at[cid, pl.ds(r0, RPT)])


def _messages(hs_pad, src_p, dst_p):
    mesh = plsc.VectorSubcoreMesh(core_axis_name="c", subcore_axis_name="s")
    return pl.kernel(
        _msg_body,
        out_type=jax.ShapeDtypeStruct((NC, NPAD, D), jnp.float32),
        mesh=mesh,
        scratch_types=[
            pltpu.VMEM((HCH, CHUNK), jnp.int32),
            pltpu.VMEM((HCH, CHUNK), jnp.int32),
            pltpu.VMEM((NBUF, CHUNK, D), jnp.float32),
            pltpu.VMEM_SHARED((NPAD, D), jnp.float32),
            pltpu.SemaphoreType.DMA,
            pltpu.SemaphoreType.DMA,
            pltpu.SemaphoreType.DMA,
            pltpu.SemaphoreType.DMA,
        ],
        compiler_params=_sc_params(),
    )(hs_pad, src_p, dst_p)


def _fin_body(part_ref, h0_ref, nd_ref, nsnd_ref, w_ref, b_ref, out_ref):
    p = part_ref[...]
    h0 = h0_ref[...]
    a1 = (p[0] + p[1]) * nd_ref[...][:, None] + h0 * nsnd_ref[...][:, None]
    h1 = jnp.dot(a1, w_ref[...], preferred_element_type=jnp.float32,
                 precision=_HIGH)
    out_ref[...] = h0 + h1 + b_ref[...][None, :]


def _finalize(part, h0, nd, nsnd, w_gc, b_gc):
    grid = (NPAD // BR,)
    return pl.pallas_call(
        _fin_body,
        grid=grid,
        in_specs=[
            pl.BlockSpec((NC, BR, D), lambda i: (0, i, 0)),
            pl.BlockSpec((BR, D), lambda i: (i, 0)),
            pl.BlockSpec((BR,), lambda i: (i,)),
            pl.BlockSpec((BR,), lambda i: (i,)),
            pl.BlockSpec((D, D), lambda i: (0, 0)),
            pl.BlockSpec((D,), lambda i: (0,)),
        ],
        out_specs=pl.BlockSpec((BR, D), lambda i: (i, 0)),
        out_shape=jax.ShapeDtypeStruct((N, D), jnp.float32),
    )(part, h0, nd, nsnd, w_gc, b_gc)


def kernel(node_features, edge_index, edgetypes, W_enc, b_enc, W_gc, b_gc):
    del edgetypes
    dummy = (N + jnp.arange(EPT - E // NW, dtype=jnp.int32))[None, :]
    dummy = jnp.broadcast_to(dummy, (NW, EPT - E // NW))
    src_p = jnp.concatenate(
        [edge_index[0].reshape(NW, E // NW), dummy], axis=1
    ).reshape(NW, NHALF, HCH, CHUNK)
    dst_p = jnp.concatenate(
        [edge_index[1].reshape(NW, E // NW), dummy], axis=1
    ).reshape(NW, NHALF, HCH, CHUNK)

    hist = _degrees(src_p, dst_p)
    h0 = _ench0(node_features, W_enc, b_enc)
    hs_pad, nd, nsnd = _scale(h0, hist)
    part = _messages(hs_pad, src_p, dst_p)
    return _finalize(part, h0, nd, nsnd, W_gc, b_gc)


# speedup vs baseline: 16.9208x; 1.1150x over previous
import dataclasses

import jax
import jax.numpy as jnp
from jax import lax
from jax.experimental import pallas as pl
from jax.experimental.pallas import tpu as pltpu
from jax.experimental.pallas import tpu_sc as plsc

N = 10000
E = 320000
D = 128

NC = 2
NS = 16
NW = NC * NS

CHUNK = 128
NCHT = E // CHUNK
CPT = NCHT // NW
NXTRA = NCHT - CPT * NW
NHALF = 2
HCH = CPT // NHALF
RPT = 624
RREM = N - RPT * NS

BR = 1024
_PREC = jax.lax.Precision.DEFAULT
NBUF = 2


def _sc_params():
    cp = pltpu.CompilerParams()
    if "needs_layout_passes" in pltpu.CompilerParams.__dataclass_fields__:
        cp = dataclasses.replace(cp, needs_layout_passes=False)
    return cp


def _deg_body(ei_hbm, hist_hbm, ei_v, xi_v, hsrc_v, hdst_v):
    cid = lax.axis_index("c")
    sid = lax.axis_index("s")
    wid = cid * NS + sid

    pltpu.sync_copy(
        ei_hbm.at[pl.ds(0, 2), pl.ds(wid * (CPT * CHUNK), CPT * CHUNK)], ei_v)

    zeros = jnp.zeros((16,), jnp.float32)

    @pl.loop(0, N, step=16)
    def _(i):
        hsrc_v[pl.ds(i, 16)] = zeros
        hdst_v[pl.ds(i, 16)] = zeros

    ones = jnp.ones((16,), jnp.float32)

    @pl.loop(0, CPT * CHUNK, step=16)
    def _(j):
        plsc.addupdate_scatter(hsrc_v, [ei_v[0, pl.ds(j, 16)]], ones)
        plsc.addupdate_scatter(hdst_v, [ei_v[1, pl.ds(j, 16)]], ones)

    @pl.when(wid < NXTRA)
    def _():
        pltpu.sync_copy(
            ei_hbm.at[pl.ds(0, 2),
                      pl.ds((CPT * NW + wid) * CHUNK, CHUNK)], xi_v)

        @pl.loop(0, CHUNK, step=16)
        def _(j):
            plsc.addupdate_scatter(hsrc_v, [xi_v[0, pl.ds(j, 16)]], ones)
            plsc.addupdate_scatter(hdst_v, [xi_v[1, pl.ds(j, 16)]], ones)

    pltpu.sync_copy(hsrc_v, hist_hbm.at[wid, 0])
    pltpu.sync_copy(hdst_v, hist_hbm.at[wid, 1])


def _degrees(ei):
    mesh = plsc.VectorSubcoreMesh(core_axis_name="c", subcore_axis_name="s")
    return pl.kernel(
        _deg_body,
        out_type=jax.ShapeDtypeStruct((NW, 2, N), jnp.float32),
        mesh=mesh,
        scratch_types=[
            pltpu.VMEM((2, CPT * CHUNK), jnp.int32),
            pltpu.VMEM((2, CHUNK), jnp.int32),
            pltpu.VMEM((N,), jnp.float32),
            pltpu.VMEM((N,), jnp.float32),
        ],
        compiler_params=_sc_params(),
    )(ei)


def _enc_body(x_ref, w_ref, b_ref, h0_ref):
    h0 = jnp.dot(x_ref[...], w_ref[...], preferred_element_type=jnp.float32,
                 precision=_PREC)
    h0_ref[...] = jnp.maximum(h0 + b_ref[...][None, :], 0.0)


def _ench0(x, w_enc, b_enc):
    grid = ((N + BR - 1) // BR,)
    return pl.pallas_call(
        _enc_body,
        grid=grid,
        in_specs=[
            pl.BlockSpec((BR, D), lambda i: (i, 0)),
            pl.BlockSpec((D, D), lambda i: (0, 0)),
            pl.BlockSpec((D,), lambda i: (0,)),
        ],
        out_specs=pl.BlockSpec((BR, D), lambda i: (i, 0)),
        out_shape=jax.ShapeDtypeStruct((N, D), jnp.float32),
    )(x, w_enc, b_enc)


def _scale_body(h0_ref, hist_ref, hs_ref, nd_ref, nsnd_ref):
    degs = jnp.sum(hist_ref[...], axis=0) + 1.0
    ns = jax.lax.rsqrt(degs[0])
    nd = jax.lax.rsqrt(degs[1])
    hs_ref[...] = h0_ref[...] * ns[:, None]
    nd_ref[...] = nd
    nsnd_ref[...] = ns * nd


def _scale(h0, hist):
    grid = ((N + BR - 1) // BR,)
    return pl.pallas_call(
        _scale_body,
        grid=grid,
        in_specs=[
            pl.BlockSpec((BR, D), lambda i: (i, 0)),
            pl.BlockSpec((NW, 2, BR), lambda i: (0, 0, i)),
        ],
        out_specs=[
            pl.BlockSpec((BR, D), lambda i: (i, 0)),
            pl.BlockSpec((BR,), lambda i: (i,)),
            pl.BlockSpec((BR,), lambda i: (i,)),
        ],
        out_shape=[
            jax.ShapeDtypeStruct((N, D), jnp.float32),
            jax.ShapeDtypeStruct((N,), jnp.float32),
            jax.ShapeDtypeStruct((N,), jnp.float32),
        ],
    )(h0, hist)


def _msg_body(hs_hbm, ei_hbm, part_hbm, ei_v, xi_v, buf_v, agg_sh,
              gs0, gs1, ss0, ss1):
    gsems = (gs0, gs1)
    ssems = (ss0, ss1)
    cid = lax.axis_index("c")
    sid = lax.axis_index("s")
    wid = cid * NS + sid

    zeros = jnp.zeros((16,), jnp.float32)

    @pl.loop(0, CHUNK)
    def _(r):
        @pl.loop(0, D, step=16)
        def _(j):
            buf_v[0, r, pl.ds(j, 16)] = zeros

    r0 = sid * RPT
    for k in range(RPT // CHUNK):
        pltpu.sync_copy(buf_v.at[0], agg_sh.at[pl.ds(r0 + k * CHUNK, CHUNK)])
    rem = RPT - (RPT // CHUNK) * CHUNK
    if rem:
        pltpu.sync_copy(buf_v.at[0, pl.ds(0, rem)],
                        agg_sh.at[pl.ds(r0 + RPT - rem, rem)])

    @pl.when(sid == 0)
    def _():
        pltpu.sync_copy(buf_v.at[0, pl.ds(0, RREM)],
                        agg_sh.at[pl.ds(RPT * NS, RREM)])

    plsc.subcore_barrier()

    def _gather(idx, cc, b):
        return pltpu.make_async_copy(
            hs_hbm.at[idx.at[0, pl.ds(cc * CHUNK, CHUNK)]],
            buf_v.at[b], gsems[b])

    def _scatter(idx, cc, b):
        return pltpu.make_async_copy(
            buf_v.at[b],
            agg_sh.at[idx.at[1, pl.ds(cc * CHUNK, CHUNK)]], ssems[b])

    for half in range(NHALF):
        c0 = wid * CPT + half * HCH
        pltpu.sync_copy(
            ei_hbm.at[pl.ds(0, 2), pl.ds(c0 * CHUNK, HCH * CHUNK)], ei_v)

        for b in range(NBUF):
            _gather(ei_v, b, b).start()

        @pl.loop(0, HCH - 3, step=NBUF)
        def _(c):
            for b in range(NBUF):
                cc = c + b
                _gather(ei_v, cc, b).wait()
                _scatter(ei_v, cc, b).start(add=True)
                _scatter(ei_v, cc, b).wait()
                _gather(ei_v, cc + NBUF, b).start()

        for cc in (HCH - 3, HCH - 2, HCH - 1):
            b = cc % NBUF
            _gather(ei_v, cc, b).wait()
            _scatter(ei_v, cc, b).start(add=True)
            _scatter(ei_v, cc, b).wait()
            if cc == HCH - 3:
                _gather(ei_v, cc + NBUF, b).start()

    @pl.when(wid < NXTRA)
    def _():
        pltpu.sync_copy(
            ei_hbm.at[pl.ds(0, 2), pl.ds((CPT * NW + wid) * CHUNK, CHUNK)],
            xi_v)
        pltpu.sync_copy(hs_hbm.at[xi_v.at[0]], buf_v.at[0])
        pltpu.sync_copy(buf_v.at[0], agg_sh.at[xi_v.at[1]], add=True)

    plsc.subcore_barrier()

    pltpu.sync_copy(agg_sh.at[pl.ds(r0, RPT)], part_hbm.at[cid, pl.ds(r0, RPT)])

    @pl.when(sid == 0)
    def _():
        pltpu.sync_copy(agg_sh.at[pl.ds(RPT * NS, RREM)],
                        part_hbm.at[cid, pl.ds(RPT * NS, RREM)])


def _messages(hs, ei):
    mesh = plsc.VectorSubcoreMesh(core_axis_name="c", subcore_axis_name="s")
    return pl.kernel(
        _msg_body,
        out_type=jax.ShapeDtypeStruct((NC, N, D), jnp.float32),
        mesh=mesh,
        scratch_types=[
            pltpu.VMEM((2, HCH * CHUNK), jnp.int32),
            pltpu.VMEM((2, CHUNK), jnp.int32),
            pltpu.VMEM((NBUF, CHUNK, D), jnp.float32),
            pltpu.VMEM_SHARED((N, D), jnp.float32),
            pltpu.SemaphoreType.DMA,
            pltpu.SemaphoreType.DMA,
            pltpu.SemaphoreType.DMA,
            pltpu.SemaphoreType.DMA,
        ],
        compiler_params=_sc_params(),
    )(hs, ei)


def _fin_body(part_ref, h0_ref, nd_ref, nsnd_ref, w_ref, b_ref, out_ref):
    p = part_ref[...]
    h0 = h0_ref[...]
    a1 = (p[0] + p[1]) * nd_ref[...][:, None] + h0 * nsnd_ref[...][:, None]
    h1 = jnp.dot(a1, w_ref[...], preferred_element_type=jnp.float32,
                 precision=_PREC)
    out_ref[...] = h0 + h1 + b_ref[...][None, :]


def _finalize(part, h0, nd, nsnd, w_gc, b_gc):
    grid = ((N + BR - 1) // BR,)
    return pl.pallas_call(
        _fin_body,
        grid=grid,
        in_specs=[
            pl.BlockSpec((NC, BR, D), lambda i: (0, i, 0)),
            pl.BlockSpec((BR, D), lambda i: (i, 0)),
            pl.BlockSpec((BR,), lambda i: (i,)),
            pl.BlockSpec((BR,), lambda i: (i,)),
            pl.BlockSpec((D, D), lambda i: (0, 0)),
            pl.BlockSpec((D,), lambda i: (0,)),
        ],
        out_specs=pl.BlockSpec((BR, D), lambda i: (i, 0)),
        out_shape=jax.ShapeDtypeStruct((N, D), jnp.float32),
    )(part, h0, nd, nsnd, w_gc, b_gc)


def kernel(node_features, edge_index, edgetypes, W_enc, b_enc, W_gc, b_gc):
    del edgetypes
    hist = _degrees(edge_index)
    h0 = _ench0(node_features, W_enc, b_enc)
    hs, nd, nsnd = _scale(h0, hist)
    part = _messages(hs, edge_index)
    return _finalize(part, h0, nd, nsnd, W_gc, b_gc)
